# baseline (speedup 1.0000x reference)
import jax
import jax.numpy as jnp
from jax import lax
from jax.experimental import pallas as pl
from jax.experimental.pallas import tpu as pltpu
from jax.experimental.pallas import tpu_sc as plsc

_NAMES = ["f%d" % i for i in range(26)] + ["grp"]
_NF = 27
_NPLAIN = 26
_GRP_COLS = 4
_B = 4096
_EMB = 64
_NC = 2
_NS = 16
_NW = _NC * _NS
_BPW = _B // _NW
_NBUF = 4


def _body(obs_hbm, *refs):
    tabs = refs[:_NF]
    outs = refs[_NF:2 * _NF]
    scratch = refs[2 * _NF:]
    idx_v = scratch[0]
    grp_v = scratch[1]
    acc_v = scratch[2]
    rows = scratch[3:3 + _NBUF]
    gsems = scratch[3 + _NBUF:3 + 2 * _NBUF]
    wsems = scratch[3 + 2 * _NBUF:3 + 3 * _NBUF]
    gsem2 = scratch[3 + 3 * _NBUF]

    wid = lax.axis_index("s") * _NC + lax.axis_index("c")
    base = wid * _BPW

    pltpu.sync_copy(obs_hbm.at[:, pl.ds(base, _BPW)], idx_v)

    wcopies = [None] * _NBUF
    for f in range(_NPLAIN):
        b = f % _NBUF
        if wcopies[b] is not None:
            wcopies[b].wait()
        pltpu.async_copy(tabs[f].at[idx_v.at[f]], rows[b], gsems[b]).wait()
        wcopies[b] = pltpu.async_copy(rows[b], outs[f].at[pl.ds(base, _BPW)],
                                      wsems[b])

    gcopies = []
    for j in range(_GRP_COLS):
        gcopies.append(
            pltpu.async_copy(tabs[_NF - 1].at[idx_v.at[j]],
                             grp_v.at[pl.ds(j * _BPW, _BPW)], gsem2))
    for c in gcopies:
        c.wait()

    def _red(r, carry):
        for c in range(_EMB // 16):
            s0 = grp_v[r, pl.ds(c * 16, 16)]
            s1 = grp_v[r + _BPW, pl.ds(c * 16, 16)]
            s2 = grp_v[r + 2 * _BPW, pl.ds(c * 16, 16)]
            s3 = grp_v[r + 3 * _BPW, pl.ds(c * 16, 16)]
            acc_v[r, pl.ds(c * 16, 16)] = (s0 + s1) + (s2 + s3)
        return carry

    lax.fori_loop(0, _BPW, _red, 0, unroll=2)

    pltpu.sync_copy(acc_v, outs[_NF - 1].at[pl.ds(base, _BPW)])
    for c in wcopies:
        if c is not None:
            c.wait()


def kernel(observation, tables):
    obs_t = observation.T

    mesh = plsc.VectorSubcoreMesh(core_axis_name="c", subcore_axis_name="s")
    out_type = [jax.ShapeDtypeStruct((_B, _EMB), jnp.float32)] * _NF
    scratch = (
        [pltpu.VMEM((_NF, _BPW), jnp.int32),
         pltpu.VMEM((_GRP_COLS * _BPW, _EMB), jnp.float32),
         pltpu.VMEM((_BPW, _EMB), jnp.float32)]
        + [pltpu.VMEM((_BPW, _EMB), jnp.float32) for _ in range(_NBUF)]
        + [pltpu.SemaphoreType.DMA for _ in range(2 * _NBUF + 1)]
    )
    run = pl.kernel(_body, out_type=out_type, mesh=mesh,
                    scratch_types=scratch,
                    compiler_params=pltpu.CompilerParams(
                        use_tc_tiling_on_sc=False))
    outs = run(obs_t, *[tables[n] for n in _NAMES])
    return tuple(outs)


# speedup vs baseline: 1.0129x; 1.0129x over previous
import jax
import jax.numpy as jnp
from jax import lax
from jax.experimental import pallas as pl
from jax.experimental.pallas import tpu as pltpu
from jax.experimental.pallas import tpu_sc as plsc

_NAMES = ["f%d" % i for i in range(26)] + ["grp"]
_NF = 27
_NPLAIN = 26
_GRP_COLS = 4
_B = 4096
_EMB = 64
_NC = 2
_NS = 16
_NW = _NC * _NS
_BPW = _B // _NW
_NBUF = 6


def _body(obs_hbm, *refs):
    tabs = refs[:_NF]
    outs = refs[_NF:2 * _NF]
    scratch = refs[2 * _NF:]
    idx_v = scratch[0]
    grp_v = scratch[1]
    acc_v = scratch[2]
    rows = scratch[3:3 + _NBUF]
    gsems = scratch[3 + _NBUF:3 + 2 * _NBUF]
    wsems = scratch[3 + 2 * _NBUF:3 + 3 * _NBUF]
    gsem2 = scratch[3 + 3 * _NBUF]

    wid = lax.axis_index("s") * _NC + lax.axis_index("c")
    base = wid * _BPW

    pltpu.sync_copy(obs_hbm.at[:, pl.ds(base, _BPW)], idx_v)

    gcopies = []
    for j in range(_GRP_COLS):
        gcopies.append(
            pltpu.async_copy(tabs[_NF - 1].at[idx_v.at[j]],
                             grp_v.at[pl.ds(j * _BPW, _BPW)], gsem2))

    gcop = [None] * _NBUF
    wcop = [None] * _NBUF
    for f in range(min(_NBUF, _NPLAIN)):
        gcop[f] = pltpu.async_copy(tabs[f].at[idx_v.at[f]], rows[f], gsems[f])
    for f in range(_NPLAIN):
        b = f % _NBUF
        gcop[b].wait()
        wcop[b] = pltpu.async_copy(rows[b], outs[f].at[pl.ds(base, _BPW)],
                                   wsems[b])
        nf = f + _NBUF
        if nf < _NPLAIN:
            wcop[b].wait()
            wcop[b] = None
            gcop[b] = pltpu.async_copy(tabs[nf].at[idx_v.at[nf]], rows[b],
                                       gsems[b])

    for c in gcopies:
        c.wait()

    def _red(r, carry):
        for c in range(_EMB // 16):
            s0 = grp_v[r, pl.ds(c * 16, 16)]
            s1 = grp_v[r + _BPW, pl.ds(c * 16, 16)]
            s2 = grp_v[r + 2 * _BPW, pl.ds(c * 16, 16)]
            s3 = grp_v[r + 3 * _BPW, pl.ds(c * 16, 16)]
            acc_v[r, pl.ds(c * 16, 16)] = (s0 + s1) + (s2 + s3)
        return carry

    lax.fori_loop(0, _BPW, _red, 0, unroll=8)

    pltpu.sync_copy(acc_v, outs[_NF - 1].at[pl.ds(base, _BPW)])
    for c in wcop:
        if c is not None:
            c.wait()


def kernel(observation, tables):
    obs_t = observation.T

    mesh = plsc.VectorSubcoreMesh(core_axis_name="c", subcore_axis_name="s")
    out_type = [jax.ShapeDtypeStruct((_B, _EMB), jnp.float32)] * _NF
    scratch = (
        [pltpu.VMEM((_NF, _BPW), jnp.int32),
         pltpu.VMEM((_GRP_COLS * _BPW, _EMB), jnp.float32),
         pltpu.VMEM((_BPW, _EMB), jnp.float32)]
        + [pltpu.VMEM((_BPW, _EMB), jnp.float32) for _ in range(_NBUF)]
        + [pltpu.SemaphoreType.DMA for _ in range(2 * _NBUF + 1)]
    )
    run = pl.kernel(_body, out_type=out_type, mesh=mesh,
                    scratch_types=scratch,
                    compiler_params=pltpu.CompilerParams(
                        use_tc_tiling_on_sc=False))
    outs = run(obs_t, *[tables[n] for n in _NAMES])
    return tuple(outs)


# speedup vs baseline: 1.3787x; 1.3612x over previous
import jax
import jax.numpy as jnp
from jax import lax
from jax.experimental import pallas as pl
from jax.experimental.pallas import tpu as pltpu
from jax.experimental.pallas import tpu_sc as plsc

_NAMES = ["f%d" % i for i in range(26)] + ["grp"]
_NF = 27
_NPLAIN = 26
_GRP_COLS = 4
_B = 4096
_EMB = 64
_NC = 2
_NS = 16
_NW = _NC * _NS
_BPW = _B // _NW
_NBUF = 4
_LAG = 2


def _enqueue_field_gather(tab, idx_row, dst, sem):
    def chunk(c, carry):
        v = idx_row[pl.ds(c * 16, 16)]
        for l in range(16):
            r = v[l]
            pltpu.async_copy(tab.at[pl.ds(r, 1)],
                             dst.at[pl.ds(c * 16 + l, 1)], sem)
        return carry

    lax.fori_loop(0, _BPW // 16, chunk, 0)


def _drain(tab, dst, sem):
    pltpu.make_async_copy(tab.at[pl.ds(0, _BPW)], dst, sem).wait()


def _body(obs_hbm, *refs):
    tabs = refs[:_NF]
    outs = refs[_NF:2 * _NF]
    scratch = refs[2 * _NF:]
    idx_v = scratch[0]
    acc_v = scratch[1]
    rows = scratch[2:2 + _NBUF]
    gsems = scratch[2 + _NBUF:2 + 2 * _NBUF]
    wsems = scratch[2 + 2 * _NBUF:2 + 3 * _NBUF]

    wid = lax.axis_index("s") * _NC + lax.axis_index("c")
    base = wid * _BPW

    pltpu.sync_copy(obs_hbm.at[:, pl.ds(base, _BPW)], idx_v)

    wcop = [None] * _NBUF
    nfields = _NPLAIN + _GRP_COLS
    for f in range(nfields + _LAG):
        if f < nfields:
            b = f % _NBUF
            if wcop[b] is not None:
                wcop[b].wait()
                wcop[b] = None
            tab = tabs[f] if f < _NPLAIN else tabs[_NF - 1]
            irow = f if f < _NPLAIN else f - _NPLAIN
            _enqueue_field_gather(tab, idx_v.at[irow], rows[b], gsems[b])
        d = f - _LAG
        if d >= 0:
            db = d % _NBUF
            dtab = tabs[d] if d < _NPLAIN else tabs[_NF - 1]
            _drain(dtab, rows[db], gsems[db])
            if d < _NPLAIN:
                wcop[db] = pltpu.async_copy(rows[db],
                                            outs[d].at[pl.ds(base, _BPW)],
                                            wsems[db])

    g0 = rows[26 % _NBUF]
    g1 = rows[27 % _NBUF]
    g2 = rows[28 % _NBUF]
    g3 = rows[29 % _NBUF]

    def _red(r, carry):
        for c in range(_EMB // 16):
            s0 = g0[r, pl.ds(c * 16, 16)]
            s1 = g1[r, pl.ds(c * 16, 16)]
            s2 = g2[r, pl.ds(c * 16, 16)]
            s3 = g3[r, pl.ds(c * 16, 16)]
            acc_v[r, pl.ds(c * 16, 16)] = (s0 + s1) + (s2 + s3)
        return carry

    lax.fori_loop(0, _BPW, _red, 0, unroll=4)

    pltpu.sync_copy(acc_v, outs[_NF - 1].at[pl.ds(base, _BPW)])
    for c in wcop:
        if c is not None:
            c.wait()


def kernel(observation, tables):
    obs_t = observation.T

    mesh = plsc.VectorSubcoreMesh(core_axis_name="c", subcore_axis_name="s")
    out_type = [jax.ShapeDtypeStruct((_B, _EMB), jnp.float32)] * _NF
    scratch = (
        [pltpu.VMEM((_NF, _BPW), jnp.int32),
         pltpu.VMEM((_BPW, _EMB), jnp.float32)]
        + [pltpu.VMEM((_BPW, _EMB), jnp.float32) for _ in range(_NBUF)]
        + [pltpu.SemaphoreType.DMA for _ in range(2 * _NBUF)]
    )
    run = pl.kernel(_body, out_type=out_type, mesh=mesh,
                    scratch_types=scratch)
    outs = run(obs_t, *[tables[n] for n in _NAMES])
    return tuple(outs)
